# baseline (speedup 1.0000x reference)
import functools

import jax
import jax.numpy as jnp
from jax import lax
from jax.experimental import pallas as pl
from jax.experimental.pallas import tpu as pltpu
from jax.experimental.pallas import tpu_sc as plsc

A = 46
LANES = 16
NC, NS = 2, 16
NW = NC * NS


def _tc_body(x_ref, s_ref, o_ref):
    x = x_ref[...]
    s2 = s_ref[...]
    g, l = s2.shape
    x3 = x.reshape(g, l, A)
    s3 = s2.reshape(g, l, 1)
    lane = lax.broadcasted_iota(jnp.int32, (g, l, A), 2)
    col = lane - s3
    col = jnp.where(col < 0, col + A, col)
    o_ref[...] = jnp.take_along_axis(x3, col, axis=2).reshape(g * l, A)


@functools.partial(jax.jit, static_argnames=("g_rows",))
def _tc_call(clear, shifts, g_rows):
    b, l, a = clear.shape
    n_rows = b * l
    x1 = clear.reshape(n_rows, a)
    return pl.pallas_call(
        _tc_body,
        grid=(b // g_rows,),
        in_specs=[
            pl.BlockSpec((g_rows * l, a), lambda i: (i, 0)),
            pl.BlockSpec((g_rows, l), lambda i: (i, 0)),
        ],
        out_specs=pl.BlockSpec((g_rows * l, a), lambda i: (i, 0)),
        out_shape=jax.ShapeDtypeStruct((n_rows, a), clear.dtype),
        compiler_params=pltpu.CompilerParams(
            dimension_semantics=("parallel",)),
    )(x1, shifts)


def kernel(clear, shifts):
    b, l, a = clear.shape
    return _tc_call(clear, shifts, 16).reshape(b, l, a)
